# baseline (device time: 26993 ns/iter reference)
import jax
import jax.numpy as jnp
from jax import lax
from jax.experimental import pallas as pl
from jax.experimental.pallas import tpu as pltpu

N_DEV = 4


def kernel(x, w_mat):
    m_per, k = x.shape
    _, n_per = w_mat.shape
    half = m_per // 2

    def body(x_ref, w_ref, out_ref, from_left, from_right, opp_buf,
             send_sems, recv_sems):
        my = lax.axis_index("i")
        left = (my - 1) % N_DEV
        right = (my + 1) % N_DEV

        barrier_sem = pltpu.get_barrier_semaphore()
        for nbr in (left, right):
            pl.semaphore_signal(
                barrier_sem, inc=1,
                device_id=(nbr,), device_id_type=pl.DeviceIdType.MESH,
            )
        pl.semaphore_wait(barrier_sem, 2)

        rdma_r = pltpu.make_async_remote_copy(
            src_ref=x_ref, dst_ref=from_left,
            send_sem=send_sems.at[0], recv_sem=recv_sems.at[0],
            device_id=(right,), device_id_type=pl.DeviceIdType.MESH,
        )
        rdma_l = pltpu.make_async_remote_copy(
            src_ref=x_ref, dst_ref=from_right,
            send_sem=send_sems.at[1], recv_sem=recv_sems.at[1],
            device_id=(left,), device_id_type=pl.DeviceIdType.MESH,
        )
        rdma_r.start()
        rdma_l.start()

        def block(chunk):
            y = jnp.dot(chunk, w_ref[:, :], preferred_element_type=jnp.float32)
            return y * jax.nn.sigmoid(y)

        out_ref[pl.ds(my * m_per, m_per), :] = block(x_ref[:, :])

        rdma_r.wait_recv()
        fwd_r = pltpu.make_async_remote_copy(
            src_ref=from_left.at[pl.ds(0, half), :],
            dst_ref=opp_buf.at[pl.ds(0, half), :],
            send_sem=send_sems.at[2], recv_sem=recv_sems.at[2],
            device_id=(right,), device_id_type=pl.DeviceIdType.MESH,
        )
        fwd_r.start()

        rdma_l.wait_recv()
        fwd_l = pltpu.make_async_remote_copy(
            src_ref=from_right.at[pl.ds(half, half), :],
            dst_ref=opp_buf.at[pl.ds(half, half), :],
            send_sem=send_sems.at[3], recv_sem=recv_sems.at[3],
            device_id=(left,), device_id_type=pl.DeviceIdType.MESH,
        )
        fwd_l.start()

        out_ref[pl.ds(left * m_per, m_per), :] = block(from_left[:, :])
        out_ref[pl.ds(right * m_per, m_per), :] = block(from_right[:, :])

        fwd_r.wait_recv()
        fwd_l.wait_recv()
        opp = (my + 2) % N_DEV
        out_ref[pl.ds(opp * m_per, m_per), :] = block(opp_buf[:, :])

        rdma_r.wait_send()
        rdma_l.wait_send()
        fwd_r.wait_send()
        fwd_l.wait_send()

    return pl.pallas_call(
        body,
        out_shape=jax.ShapeDtypeStruct((N_DEV * m_per, n_per), jnp.float32),
        in_specs=[
            pl.BlockSpec(memory_space=pltpu.VMEM),
            pl.BlockSpec(memory_space=pltpu.VMEM),
        ],
        out_specs=pl.BlockSpec(memory_space=pltpu.VMEM),
        scratch_shapes=[
            pltpu.VMEM((m_per, k), jnp.float32),
            pltpu.VMEM((m_per, k), jnp.float32),
            pltpu.VMEM((m_per, k), jnp.float32),
            pltpu.SemaphoreType.DMA((4,)),
            pltpu.SemaphoreType.DMA((4,)),
        ],
        compiler_params=pltpu.CompilerParams(collective_id=0),
    )(x, w_mat)


# device time: 25782 ns/iter; 1.0470x vs baseline; 1.0470x over previous
import jax
import jax.numpy as jnp
from jax import lax
from jax.experimental import pallas as pl
from jax.experimental.pallas import tpu as pltpu

N_DEV = 4


def kernel(x, w_mat):
    m_per, k = x.shape
    _, n_per = w_mat.shape
    half = m_per // 2

    def body(x_ref, w_ref, out_ref, from_left, from_right, opp_buf,
             send_sems, recv_sems):
        my = lax.axis_index("i")
        left = (my - 1) % N_DEV
        right = (my + 1) % N_DEV

        top = (pl.ds(0, half), slice(None))
        bot = (pl.ds(half, half), slice(None))

        barrier_sem = pltpu.get_barrier_semaphore()
        for nbr in (left, right):
            pl.semaphore_signal(
                barrier_sem, inc=1,
                device_id=(nbr,), device_id_type=pl.DeviceIdType.MESH,
            )
        pl.semaphore_wait(barrier_sem, 2)

        def rcopy(src, dst, i, dev):
            return pltpu.make_async_remote_copy(
                src_ref=src, dst_ref=dst,
                send_sem=send_sems.at[i], recv_sem=recv_sems.at[i],
                device_id=(dev,), device_id_type=pl.DeviceIdType.MESH,
            )

        a0 = rcopy(x_ref.at[top], from_left.at[top], 0, right)
        a1 = rcopy(x_ref.at[bot], from_left.at[bot], 1, right)
        a2 = rcopy(x_ref.at[bot], from_right.at[bot], 2, left)
        a3 = rcopy(x_ref.at[top], from_right.at[top], 3, left)
        a0.start()
        a2.start()
        a1.start()
        a3.start()

        def block(chunk):
            y = jnp.dot(chunk, w_ref[:, :], preferred_element_type=jnp.float32)
            return y * jax.nn.sigmoid(y)

        out_ref[pl.ds(my * m_per, m_per), :] = block(x_ref[:, :])

        a0.wait_recv()
        fwd_r = rcopy(from_left.at[top], opp_buf.at[top], 4, right)
        fwd_r.start()

        a2.wait_recv()
        fwd_l = rcopy(from_right.at[bot], opp_buf.at[bot], 5, left)
        fwd_l.start()

        a1.wait_recv()
        out_ref[pl.ds(left * m_per, m_per), :] = block(from_left[:, :])
        a3.wait_recv()
        out_ref[pl.ds(right * m_per, m_per), :] = block(from_right[:, :])

        fwd_r.wait_recv()
        fwd_l.wait_recv()
        opp = (my + 2) % N_DEV
        out_ref[pl.ds(opp * m_per, m_per), :] = block(opp_buf[:, :])

        for r in (a0, a1, a2, a3, fwd_r, fwd_l):
            r.wait_send()

    return pl.pallas_call(
        body,
        out_shape=jax.ShapeDtypeStruct((N_DEV * m_per, n_per), jnp.float32),
        in_specs=[
            pl.BlockSpec(memory_space=pltpu.VMEM),
            pl.BlockSpec(memory_space=pltpu.VMEM),
        ],
        out_specs=pl.BlockSpec(memory_space=pltpu.VMEM),
        scratch_shapes=[
            pltpu.VMEM((m_per, k), jnp.float32),
            pltpu.VMEM((m_per, k), jnp.float32),
            pltpu.VMEM((m_per, k), jnp.float32),
            pltpu.SemaphoreType.DMA((6,)),
            pltpu.SemaphoreType.DMA((6,)),
        ],
        compiler_params=pltpu.CompilerParams(collective_id=0),
    )(x, w_mat)
